# baseline (speedup 1.0000x reference)
import functools

import jax
import jax.numpy as jnp
from jax import lax
from jax.experimental import pallas as pl
from jax.experimental.pallas import tpu as pltpu
from jax.experimental.pallas import tpu_sc as plsc

N_ROWS = 320000
D = 128
S = 10000
NC = 2
NS = 16
DC = D // NC
ROWS_PER_SUB = N_ROWS // NS
SUB = 100
CHUNK = 400
NSUB = CHUNK // SUB
NCHUNK = ROWS_PER_SUB // CHUNK
NBUF = 2
NGRP = NCHUNK // NBUF
NIDX = ROWS_PER_SUB // SUB
SEG_PER_SUB = S // NS
ZROWS = 125


def kernel(x, molecule_idx):
    idx2d = molecule_idx.reshape(N_ROWS // SUB, SUB)

    mesh = plsc.VectorSubcoreMesh(
        core_axis_name="c", subcore_axis_name="s", num_cores=NC, num_subcores=NS
    )

    @functools.partial(
        pl.kernel,
        out_type=jax.ShapeDtypeStruct((S, D), jnp.float32),
        mesh=mesh,
        scratch_types=[
            pltpu.VMEM((NIDX, SUB), jnp.int32),
            pltpu.VMEM((NBUF * CHUNK, DC), jnp.float32),
            pltpu.VMEM_SHARED((S, DC), jnp.float32),
            pltpu.SemaphoreType.DMA((NBUF,)),
        ],
        compiler_params=pltpu.CompilerParams(use_tc_tiling_on_sc=False),
    )
    def sc_kernel(x_hbm, idx_hbm, out_hbm, idx_v, x_v, acc_sh, xsem):
        cid = lax.axis_index("c")
        sid = lax.axis_index("s")
        seg0 = sid * SEG_PER_SUB
        col0 = cid * DC
        row0 = sid * ROWS_PER_SUB
        irow0 = sid * (ROWS_PER_SUB // SUB)

        pltpu.sync_copy(idx_hbm.at[pl.ds(irow0, NIDX), :], idx_v)

        zv = jnp.zeros((16,), jnp.float32)

        def zero_row(i, carry):
            for j in range(DC // 16):
                x_v[i, pl.ds(j * 16, 16)] = zv
            return carry

        lax.fori_loop(0, ZROWS, zero_row, 0)
        for k in range(SEG_PER_SUB // ZROWS):
            pltpu.sync_copy(x_v.at[pl.ds(0, ZROWS)],
                            acc_sh.at[pl.ds(seg0 + k * ZROWS, ZROWS), :])
        plsc.subcore_barrier()

        def start_load(c, b):
            pltpu.async_copy(
                x_hbm.at[pl.ds(row0 + c * CHUNK, CHUNK), pl.ds(col0, DC)],
                x_v.at[pl.ds(b * CHUNK, CHUNK)], xsem.at[b])

        def wait_load(b):
            pltpu.make_async_copy(
                x_hbm.at[pl.ds(0, CHUNK), pl.ds(0, DC)],
                x_v.at[pl.ds(b * CHUNK, CHUNK)], xsem.at[b]).wait()

        def do_scatters(c, b):
            for j in range(NSUB):
                pltpu.sync_copy(
                    x_v.at[pl.ds(b * CHUNK + j * SUB, SUB)],
                    acc_sh.at[idx_v.at[c * NSUB + j]], add=True)

        for b in range(NBUF):
            start_load(b, b)

        def group_body(g, carry):
            for b in range(NBUF):
                c = g * NBUF + b
                wait_load(b)
                do_scatters(c, b)
                nxt = c + NBUF

                @pl.when(nxt < NCHUNK)
                def _():
                    start_load(nxt, b)
            return carry

        lax.fori_loop(0, NGRP, group_body, 0)
        plsc.subcore_barrier()

        for k in range(SEG_PER_SUB // ZROWS):
            r = seg0 + k * ZROWS
            pltpu.sync_copy(acc_sh.at[pl.ds(r, ZROWS), :],
                            x_v.at[pl.ds(0, ZROWS)])
            pltpu.sync_copy(x_v.at[pl.ds(0, ZROWS)],
                            out_hbm.at[pl.ds(r, ZROWS), pl.ds(col0, DC)])

    return sc_kernel(x, idx2d)


# speedup vs baseline: 1.0573x; 1.0573x over previous
import functools

import jax
import jax.numpy as jnp
from jax import lax
from jax.experimental import pallas as pl
from jax.experimental.pallas import tpu as pltpu
from jax.experimental.pallas import tpu_sc as plsc

N_ROWS = 320000
D = 128
S = 10000
NC = 2
NS = 16
DC = D // NC
ROWS_PER_SUB = N_ROWS // NS
SUB = 80
CHUNK = 400
NSUB = CHUNK // SUB
NCHUNK = ROWS_PER_SUB // CHUNK
NBUF = 3
SEG_PER_SUB = S // NS
ZROWS = 125


def kernel(x, molecule_idx):
    idx2d = molecule_idx.reshape(N_ROWS // SUB, SUB)

    mesh = plsc.VectorSubcoreMesh(
        core_axis_name="c", subcore_axis_name="s", num_cores=NC, num_subcores=NS
    )

    @functools.partial(
        pl.kernel,
        out_type=jax.ShapeDtypeStruct((S, D), jnp.float32),
        mesh=mesh,
        scratch_types=[
            pltpu.VMEM((NBUF * NSUB, SUB), jnp.int32),
            pltpu.VMEM((NBUF * CHUNK, DC), jnp.float32),
            pltpu.VMEM((ZROWS, DC), jnp.float32),
            pltpu.VMEM_SHARED((S, DC), jnp.float32),
            pltpu.SemaphoreType.DMA((NBUF,)),
            pltpu.SemaphoreType.DMA((NBUF,)),
        ],
        compiler_params=pltpu.CompilerParams(use_tc_tiling_on_sc=False),
    )
    def sc_kernel(x_hbm, idx_hbm, out_hbm, idx_v, x_v, stage_v, acc_sh,
                  xsem, isem):
        cid = lax.axis_index("c")
        sid = lax.axis_index("s")
        seg0 = sid * SEG_PER_SUB
        col0 = cid * DC
        row0 = sid * ROWS_PER_SUB
        irow0 = sid * (ROWS_PER_SUB // SUB)

        zv = jnp.zeros((16,), jnp.float32)

        def zero_row(i, carry):
            for j in range(DC // 16):
                stage_v[i, pl.ds(j * 16, 16)] = zv
            return carry

        lax.fori_loop(0, ZROWS, zero_row, 0)
        for k in range(SEG_PER_SUB // ZROWS):
            pltpu.sync_copy(stage_v, acc_sh.at[pl.ds(seg0 + k * ZROWS, ZROWS), :])
        plsc.subcore_barrier()

        def start_load(c, b):
            pltpu.async_copy(
                x_hbm.at[pl.ds(row0 + c * CHUNK, CHUNK), pl.ds(col0, DC)],
                x_v.at[pl.ds(b * CHUNK, CHUNK)], xsem.at[b])
            pltpu.async_copy(
                idx_hbm.at[pl.ds(irow0 + c * NSUB, NSUB), :],
                idx_v.at[pl.ds(b * NSUB, NSUB)], isem.at[b])

        def wait_load(b):
            pltpu.make_async_copy(
                x_hbm.at[pl.ds(0, CHUNK), pl.ds(0, DC)],
                x_v.at[pl.ds(b * CHUNK, CHUNK)], xsem.at[b]).wait()
            pltpu.make_async_copy(
                idx_hbm.at[pl.ds(0, NSUB), :],
                idx_v.at[pl.ds(b * NSUB, NSUB)], isem.at[b]).wait()

        for b in range(NBUF):
            start_load(b, b)


        def chunk_body(g, carry):
            for b in range(NBUF):
                c = g * NBUF + b
                wait_load(b)
                for j in range(NSUB):
                    pltpu.sync_copy(
                        x_v.at[pl.ds(b * CHUNK + j * SUB, SUB)],
                        acc_sh.at[idx_v.at[b * NSUB + j]], add=True)
                nxt = c + NBUF

                @pl.when(nxt < NCHUNK)
                def _():
                    start_load(nxt, b)
            return carry

        lax.fori_loop(0, (NCHUNK - 2) // NBUF, chunk_body, 0)
        for c, b in ((48, 0), (49, 1)):
            wait_load(b)
            for j in range(NSUB):
                pltpu.sync_copy(x_v.at[pl.ds(b * CHUNK + j * SUB, SUB)],
                                acc_sh.at[idx_v.at[b * NSUB + j]], add=True)
        plsc.subcore_barrier()

        for k in range(SEG_PER_SUB // ZROWS):
            r = seg0 + k * ZROWS
            pltpu.sync_copy(acc_sh.at[pl.ds(r, ZROWS), :], stage_v)
            pltpu.sync_copy(stage_v, out_hbm.at[pl.ds(r, ZROWS), pl.ds(col0, DC)])

    return sc_kernel(x, idx2d)


# speedup vs baseline: 1.0593x; 1.0019x over previous
import functools

import jax
import jax.numpy as jnp
from jax import lax
from jax.experimental import pallas as pl
from jax.experimental.pallas import tpu as pltpu
from jax.experimental.pallas import tpu_sc as plsc

N_ROWS = 320000
D = 128
S = 10000
NC = 2
NS = 16
DC = D // NC
ROWS_PER_SUB = N_ROWS // NS
SUB = 80
CHUNK = 400
NSUB = CHUNK // SUB
NCHUNK = ROWS_PER_SUB // CHUNK
NBUF = 3
SEG_PER_SUB = S // NS
ZROWS = 125


def kernel(x, molecule_idx):
    idx2d = molecule_idx.reshape(N_ROWS // SUB, SUB)

    mesh = plsc.VectorSubcoreMesh(
        core_axis_name="c", subcore_axis_name="s", num_cores=NC, num_subcores=NS
    )

    @functools.partial(
        pl.kernel,
        out_type=jax.ShapeDtypeStruct((S, D), jnp.float32),
        mesh=mesh,
        scratch_types=[
            pltpu.VMEM((NBUF * NSUB, SUB), jnp.int32),
            pltpu.VMEM((NBUF * CHUNK, DC), jnp.float32),
            pltpu.VMEM((ZROWS, DC), jnp.float32),
            pltpu.VMEM_SHARED((S, DC), jnp.float32),
            pltpu.SemaphoreType.DMA((NBUF,)),
            pltpu.SemaphoreType.DMA((NBUF,)),
            pltpu.SemaphoreType.DMA((2,)),
        ],
        compiler_params=pltpu.CompilerParams(use_tc_tiling_on_sc=False),
    )
    def sc_kernel(x_hbm, idx_hbm, out_hbm, idx_v, x_v, stage_v, acc_sh,
                  xsem, isem, osem):
        cid = lax.axis_index("c")
        sid = lax.axis_index("s")
        seg0 = sid * SEG_PER_SUB
        col0 = cid * DC
        row0 = sid * ROWS_PER_SUB
        irow0 = sid * (ROWS_PER_SUB // SUB)

        zv = jnp.zeros((16,), jnp.float32)

        def zero_row(i, carry):
            for j in range(DC // 16):
                stage_v[i, pl.ds(j * 16, 16)] = zv
            return carry

        lax.fori_loop(0, ZROWS, zero_row, 0)
        for k in range(SEG_PER_SUB // ZROWS):
            pltpu.sync_copy(stage_v, acc_sh.at[pl.ds(seg0 + k * ZROWS, ZROWS), :])
        plsc.subcore_barrier()

        def start_load(c, b):
            pltpu.async_copy(
                x_hbm.at[pl.ds(row0 + c * CHUNK, CHUNK), pl.ds(col0, DC)],
                x_v.at[pl.ds(b * CHUNK, CHUNK)], xsem.at[b])
            pltpu.async_copy(
                idx_hbm.at[pl.ds(irow0 + c * NSUB, NSUB), :],
                idx_v.at[pl.ds(b * NSUB, NSUB)], isem.at[b])

        def wait_load(b):
            pltpu.make_async_copy(
                x_hbm.at[pl.ds(0, CHUNK), pl.ds(0, DC)],
                x_v.at[pl.ds(b * CHUNK, CHUNK)], xsem.at[b]).wait()
            pltpu.make_async_copy(
                idx_hbm.at[pl.ds(0, NSUB), :],
                idx_v.at[pl.ds(b * NSUB, NSUB)], isem.at[b]).wait()

        for b in range(NBUF):
            start_load(b, b)

        def zero_row(i, carry):
            for j in range(DC // 16):
                stage_v[i, pl.ds(j * 16, 16)] = zv
            return carry

        lax.fori_loop(0, ZROWS, zero_row, 0)
        for k in range(SEG_PER_SUB // ZROWS):
            pltpu.sync_copy(stage_v, acc_sh.at[pl.ds(seg0 + k * ZROWS, ZROWS), :])
        plsc.subcore_barrier()


        def chunk_body(g, carry):
            for b in range(NBUF):
                c = g * NBUF + b
                wait_load(b)
                for j in range(NSUB):
                    pltpu.sync_copy(
                        x_v.at[pl.ds(b * CHUNK + j * SUB, SUB)],
                        acc_sh.at[idx_v.at[b * NSUB + j]], add=True)
                nxt = c + NBUF

                @pl.when(nxt < NCHUNK)
                def _():
                    start_load(nxt, b)
            return carry

        lax.fori_loop(0, (NCHUNK - 2) // NBUF, chunk_body, 0)
        for c, b in ((48, 0), (49, 1)):
            wait_load(b)
            for j in range(NSUB):
                pltpu.sync_copy(x_v.at[pl.ds(b * CHUNK + j * SUB, SUB)],
                                acc_sh.at[idx_v.at[b * NSUB + j]], add=True)
        plsc.subcore_barrier()

        def stg(s):
            return stage_v if s == 0 else x_v.at[pl.ds(0, ZROWS)]

        NK = SEG_PER_SUB // ZROWS
        pltpu.async_copy(acc_sh.at[pl.ds(seg0, ZROWS), :], stg(0), osem.at[0])
        for k in range(NK):
            s = k % 2
            pltpu.make_async_copy(acc_sh.at[pl.ds(seg0, ZROWS), :],
                                  stg(s), osem.at[s]).wait()
            if k + 1 < NK:
                pltpu.async_copy(
                    acc_sh.at[pl.ds(seg0 + (k + 1) * ZROWS, ZROWS), :],
                    stg(1 - s), osem.at[1 - s])
            r = seg0 + k * ZROWS
            pltpu.sync_copy(stg(s), out_hbm.at[pl.ds(r, ZROWS), pl.ds(col0, DC)])

    return sc_kernel(x, idx2d)


# speedup vs baseline: 1.0856x; 1.0248x over previous
import functools

import jax
import jax.numpy as jnp
from jax import lax
from jax.experimental import pallas as pl
from jax.experimental.pallas import tpu as pltpu
from jax.experimental.pallas import tpu_sc as plsc

N_ROWS = 320000
D = 128
S = 10000
NC = 2
NS = 16
DC = D // NC
ROWS_PER_SUB = N_ROWS // NS
SUB = 80
CHUNK = 400
NSUB = CHUNK // SUB
NCHUNK = ROWS_PER_SUB // CHUNK
NBUF = 3
SEG_PER_SUB = S // NS
ZROWS = 125


def kernel(x, molecule_idx):
    idx2d = molecule_idx.reshape(N_ROWS // SUB, SUB)

    mesh = plsc.VectorSubcoreMesh(
        core_axis_name="c", subcore_axis_name="s", num_cores=NC, num_subcores=NS
    )

    @functools.partial(
        pl.kernel,
        out_type=jax.ShapeDtypeStruct((S, D), jnp.float32),
        mesh=mesh,
        scratch_types=[
            pltpu.VMEM((NBUF * NSUB, SUB), jnp.int32),
            pltpu.VMEM((NBUF * CHUNK, DC), jnp.float32),
            pltpu.VMEM((ZROWS, DC), jnp.float32),
            pltpu.VMEM_SHARED((S, DC), jnp.float32),
            pltpu.SemaphoreType.DMA((NBUF,)),
            pltpu.SemaphoreType.DMA((NBUF,)),
            pltpu.SemaphoreType.DMA((2,)),
        ],
        compiler_params=pltpu.CompilerParams(use_tc_tiling_on_sc=False),
    )
    def sc_kernel(x_hbm, idx_hbm, out_hbm, idx_v, x_v, stage_v, acc_sh,
                  xsem, isem, osem):
        cid = lax.axis_index("c")
        sid = lax.axis_index("s")
        seg0 = sid * SEG_PER_SUB
        col0 = cid * DC
        row0 = sid * ROWS_PER_SUB
        irow0 = sid * (ROWS_PER_SUB // SUB)

        def start_load(c, b):
            pltpu.async_copy(
                x_hbm.at[pl.ds(row0 + c * CHUNK, CHUNK), pl.ds(col0, DC)],
                x_v.at[pl.ds(b * CHUNK, CHUNK)], xsem.at[b])
            pltpu.async_copy(
                idx_hbm.at[pl.ds(irow0 + c * NSUB, NSUB), :],
                idx_v.at[pl.ds(b * NSUB, NSUB)], isem.at[b])

        def wait_load(b):
            pltpu.make_async_copy(
                x_hbm.at[pl.ds(0, CHUNK), pl.ds(0, DC)],
                x_v.at[pl.ds(b * CHUNK, CHUNK)], xsem.at[b]).wait()
            pltpu.make_async_copy(
                idx_hbm.at[pl.ds(0, NSUB), :],
                idx_v.at[pl.ds(b * NSUB, NSUB)], isem.at[b]).wait()

        for b in range(NBUF):
            start_load(b, b)

        zv = jnp.zeros((16,), jnp.float32)

        def zero_row(i, carry):
            for j in range(DC // 16):
                stage_v[i, pl.ds(j * 16, 16)] = zv
            return carry

        lax.fori_loop(0, ZROWS, zero_row, 0)
        for k in range(SEG_PER_SUB // ZROWS):
            pltpu.sync_copy(stage_v, acc_sh.at[pl.ds(seg0 + k * ZROWS, ZROWS), :])
        plsc.subcore_barrier()


        def chunk_body(g, carry):
            for b in range(NBUF):
                c = g * NBUF + b
                wait_load(b)
                for j in range(NSUB):
                    pltpu.sync_copy(
                        x_v.at[pl.ds(b * CHUNK + j * SUB, SUB)],
                        acc_sh.at[idx_v.at[b * NSUB + j]], add=True)
                nxt = c + NBUF

                @pl.when(nxt < NCHUNK)
                def _():
                    start_load(nxt, b)
            return carry

        lax.fori_loop(0, (NCHUNK - 2) // NBUF, chunk_body, 0)
        for c, b in ((48, 0), (49, 1)):
            wait_load(b)
            for j in range(NSUB):
                pltpu.sync_copy(x_v.at[pl.ds(b * CHUNK + j * SUB, SUB)],
                                acc_sh.at[idx_v.at[b * NSUB + j]], add=True)
        plsc.subcore_barrier()

        def stg(s):
            return stage_v if s == 0 else x_v.at[pl.ds(0, ZROWS)]

        NK = SEG_PER_SUB // ZROWS
        pltpu.async_copy(acc_sh.at[pl.ds(seg0, ZROWS), :], stg(0), osem.at[0])
        for k in range(NK):
            s = k % 2
            pltpu.make_async_copy(acc_sh.at[pl.ds(seg0, ZROWS), :],
                                  stg(s), osem.at[s]).wait()
            if k + 1 < NK:
                pltpu.async_copy(
                    acc_sh.at[pl.ds(seg0 + (k + 1) * ZROWS, ZROWS), :],
                    stg(1 - s), osem.at[1 - s])
            r = seg0 + k * ZROWS
            pltpu.sync_copy(stg(s), out_hbm.at[pl.ds(r, ZROWS), pl.ds(col0, DC)])

    return sc_kernel(x, idx2d)
